# baseline (speedup 1.0000x reference)
import functools

import jax
import jax.numpy as jnp
from jax import lax
from jax.experimental import pallas as pl
from jax.experimental.pallas import tpu as pltpu
from jax.experimental.pallas import tpu_sc as plsc

N = 10000
E = 320000
D = 128
N_PAD = 10240
N_TILES = 32
CHUNK = 128
CPT = 80
E_PAD = N_TILES * CPT * CHUNK
N_STAGES = 2
CPS = CPT // N_STAGES
DUMMY = N
RPT = N_PAD // 16
DEG_W = 128
BM = 512
GRID = N_PAD // BM


def _deg_body(dst_hbm, out_hbm, dst_v, ones_v, acc):
    c = lax.axis_index("c")
    s = lax.axis_index("s")
    wid = c * 16 + s
    r0 = s * RPT

    def fill0(i, carry):
        ones_v[i] = jnp.zeros((DEG_W,), jnp.float32)
        return carry

    lax.fori_loop(0, CHUNK, fill0, 0)
    for t in range(RPT // CHUNK):
        pltpu.sync_copy(ones_v, acc.at[pl.ds(r0 + t * CHUNK, CHUNK)])

    def fill1(i, carry):
        ones_v[i] = jnp.full((DEG_W,), 1.0, jnp.float32)
        return carry

    lax.fori_loop(0, CHUNK, fill1, 0)
    plsc.subcore_barrier()

    for st in range(N_STAGES):
        pltpu.sync_copy(dst_hbm.at[wid, st], dst_v)

        def body(j, carry):
            pltpu.sync_copy(ones_v, acc.at[dst_v.at[j]], add=True)
            return carry

        lax.fori_loop(0, CPS, body, 0)
    plsc.subcore_barrier()
    pltpu.sync_copy(acc.at[pl.ds(r0, RPT)], out_hbm.at[c, pl.ds(r0, RPT)])


@functools.cache
def _deg_call():
    return pl.kernel(
        _deg_body,
        out_type=jax.ShapeDtypeStruct((2, N_PAD, DEG_W), jnp.float32),
        scratch_types=[
            pltpu.VMEM((CPS, CHUNK), jnp.int32),
            pltpu.VMEM((CHUNK, DEG_W), jnp.float32),
            pltpu.VMEM_SHARED((N_PAD, DEG_W), jnp.float32),
        ],
        mesh=plsc.VectorSubcoreMesh(core_axis_name="c", subcore_axis_name="s"),
    )


def _agg_body(g_hbm, src_hbm, dst_hbm, zeros_hbm, out_hbm,
              src_v, dst_v, rows_a, rows_b, sem_ga, sem_gb, sem_sa, sem_sb,
              acc):
    c = lax.axis_index("c")
    s = lax.axis_index("s")
    wid = c * 16 + s
    r0 = s * RPT
    pltpu.sync_copy(zeros_hbm.at[pl.ds(r0, RPT)], acc.at[pl.ds(r0, RPT)])
    plsc.subcore_barrier()

    npairs = CPS // 2

    for st in range(N_STAGES):
        pltpu.sync_copy(src_hbm.at[wid, st], src_v)
        pltpu.sync_copy(dst_hbm.at[wid, st], dst_v)
        pltpu.async_copy(g_hbm.at[src_v.at[0]], rows_a, sem_ga)
        pltpu.async_copy(g_hbm.at[src_v.at[1]], rows_b, sem_gb)

        def body(gidx, carry):
            ja = 2 * gidx
            jb = ja + 1
            pltpu.make_async_copy(g_hbm.at[src_v.at[ja]], rows_a, sem_ga).wait()
            pltpu.async_copy(rows_a, acc.at[dst_v.at[ja]], sem_sa, add=True)
            pltpu.make_async_copy(g_hbm.at[src_v.at[jb]], rows_b, sem_gb).wait()
            pltpu.async_copy(rows_b, acc.at[dst_v.at[jb]], sem_sb, add=True)

            @pl.when(gidx + 1 < npairs)
            def _refill():
                pltpu.make_async_copy(rows_a, acc.at[dst_v.at[ja]],
                                      sem_sa).wait()
                pltpu.async_copy(g_hbm.at[src_v.at[ja + 2]], rows_a, sem_ga)
                pltpu.make_async_copy(rows_b, acc.at[dst_v.at[jb]],
                                      sem_sb).wait()
                pltpu.async_copy(g_hbm.at[src_v.at[jb + 2]], rows_b, sem_gb)

            return carry

        lax.fori_loop(0, npairs, body, 0)
        pltpu.make_async_copy(rows_a, acc.at[dst_v.at[0]], sem_sa).wait()
        pltpu.make_async_copy(rows_b, acc.at[dst_v.at[1]], sem_sb).wait()
    plsc.subcore_barrier()
    pltpu.sync_copy(acc.at[pl.ds(r0, RPT)], out_hbm.at[c, pl.ds(r0, RPT)])


@functools.cache
def _agg_call():
    return pl.kernel(
        _agg_body,
        out_type=jax.ShapeDtypeStruct((2, N_PAD, D), jnp.float32),
        scratch_types=[
            pltpu.VMEM((CPS, CHUNK), jnp.int32),
            pltpu.VMEM((CPS, CHUNK), jnp.int32),
            pltpu.VMEM((CHUNK, D), jnp.float32),
            pltpu.VMEM((CHUNK, D), jnp.float32),
            pltpu.SemaphoreType.DMA,
            pltpu.SemaphoreType.DMA,
            pltpu.SemaphoreType.DMA,
            pltpu.SemaphoreType.DMA,
            pltpu.VMEM_SHARED((N_PAD, D), jnp.float32),
        ],
        mesh=plsc.VectorSubcoreMesh(core_axis_name="c", subcore_axis_name="s"),
    )


def _dinv_body(deg_ref, out_ref):
    d = deg_ref[0] + deg_ref[1]
    s = d[:, 0:1] + 1.0
    out_ref[...] = jnp.broadcast_to(lax.rsqrt(s), (N_PAD, D))


_dinv_call = pl.pallas_call(
    _dinv_body,
    out_shape=jax.ShapeDtypeStruct((N_PAD, D), jnp.float32),
)


def _mm1_body(x_ref, w_ref, dinv_ref, o_ref):
    o_ref[...] = dinv_ref[...] * jnp.dot(
        x_ref[...], w_ref[...], preferred_element_type=jnp.float32)


_mm1_call = pl.pallas_call(
    _mm1_body,
    grid=(GRID,),
    in_specs=[
        pl.BlockSpec((BM, D), lambda i: (i, 0)),
        pl.BlockSpec((D, D), lambda i: (0, 0)),
        pl.BlockSpec((BM, D), lambda i: (i, 0)),
    ],
    out_specs=pl.BlockSpec((BM, D), lambda i: (i, 0)),
    out_shape=jax.ShapeDtypeStruct((N_PAD, D), jnp.float32),
)


def _mid_body(a0_ref, a1_ref, g1_ref, dinv_ref, b1_ref, w2_ref, o_ref):
    t = dinv_ref[...] * (a0_ref[...] + a1_ref[...] + g1_ref[...]) + b1_ref[...]
    t = jnp.maximum(t, 0.0)
    o_ref[...] = dinv_ref[...] * jnp.dot(
        t, w2_ref[...], preferred_element_type=jnp.float32)


_mid_call = pl.pallas_call(
    _mid_body,
    grid=(GRID,),
    in_specs=[
        pl.BlockSpec((BM, D), lambda i: (i, 0)),
        pl.BlockSpec((BM, D), lambda i: (i, 0)),
        pl.BlockSpec((BM, D), lambda i: (i, 0)),
        pl.BlockSpec((BM, D), lambda i: (i, 0)),
        pl.BlockSpec((1, D), lambda i: (0, 0)),
        pl.BlockSpec((D, D), lambda i: (0, 0)),
    ],
    out_specs=pl.BlockSpec((BM, D), lambda i: (i, 0)),
    out_shape=jax.ShapeDtypeStruct((N_PAD, D), jnp.float32),
)


def _fin_body(a0_ref, a1_ref, g2_ref, dinv_ref, b2_ref, o_ref):
    o_ref[...] = (dinv_ref[...] * (a0_ref[...] + a1_ref[...] + g2_ref[...])
                  + b2_ref[...])


_fin_call = pl.pallas_call(
    _fin_body,
    grid=(GRID,),
    in_specs=[
        pl.BlockSpec((BM, D), lambda i: (i, 0)),
        pl.BlockSpec((BM, D), lambda i: (i, 0)),
        pl.BlockSpec((BM, D), lambda i: (i, 0)),
        pl.BlockSpec((BM, D), lambda i: (i, 0)),
        pl.BlockSpec((1, D), lambda i: (0, 0)),
    ],
    out_specs=pl.BlockSpec((BM, D), lambda i: (i, 0)),
    out_shape=jax.ShapeDtypeStruct((N_PAD, D), jnp.float32),
)


def kernel(x, edge_index, W1, b1, W2, b2):
    ei = edge_index.astype(jnp.int32)
    pad_e = E_PAD - E
    it = jnp.arange(pad_e, dtype=jnp.int32)
    src_p = jnp.concatenate(
        [ei[0], it % N]
    ).reshape(N_TILES, N_STAGES, CPS, CHUNK)
    dst_p = jnp.concatenate(
        [ei[1], DUMMY + it % (N_PAD - N)]
    ).reshape(N_TILES, N_STAGES, CPS, CHUNK)
    x_p = jnp.pad(x, ((0, N_PAD - N), (0, 0)))
    zeros_d = jnp.zeros((N_PAD, D), jnp.float32)

    deg = _deg_call()(dst_p)
    dinv = _dinv_call(deg)
    g1 = _mm1_call(x_p, W1, dinv)
    agg1 = _agg_call()(g1, src_p, dst_p, zeros_d)
    g2 = _mid_call(agg1[0], agg1[1], g1, dinv, b1.reshape(1, D), W2)
    agg2 = _agg_call()(g2, src_p, dst_p, zeros_d)
    out = _fin_call(agg2[0], agg2[1], g2, dinv, b2.reshape(1, D))
    return out[:N]


# speedup vs baseline: 1.2753x; 1.2753x over previous
import functools

import jax
import jax.numpy as jnp
from jax import lax
from jax.experimental import pallas as pl
from jax.experimental.pallas import tpu as pltpu
from jax.experimental.pallas import tpu_sc as plsc

N = 10000
E = 320000
D = 128
N_PAD = 10240
N_TILES = 32
CHUNK = 128
CPT = 80
E_PAD = N_TILES * CPT * CHUNK
N_STAGES = 2
CPS = CPT // N_STAGES
DUMMY = N
RPT = N_PAD // 16
DEG_W = 128
BM = 512
GRID = N_PAD // BM


def _deg_body(dst_hbm, out_hbm, dst_v, ones_v, acc):
    c = lax.axis_index("c")
    s = lax.axis_index("s")
    wid = c * 16 + s
    r0 = s * RPT

    def fill0(i, carry):
        ones_v[i] = jnp.zeros((DEG_W,), jnp.float32)
        return carry

    lax.fori_loop(0, CHUNK, fill0, 0)
    for t in range(RPT // CHUNK):
        pltpu.sync_copy(ones_v, acc.at[pl.ds(r0 + t * CHUNK, CHUNK)])

    def fill1(i, carry):
        ones_v[i] = jnp.full((DEG_W,), 1.0, jnp.float32)
        return carry

    lax.fori_loop(0, CHUNK, fill1, 0)
    plsc.subcore_barrier()

    for st in range(N_STAGES):
        pltpu.sync_copy(dst_hbm.at[wid, st], dst_v)

        def body(j, carry):
            pltpu.sync_copy(ones_v, acc.at[dst_v.at[j]], add=True)
            return carry

        lax.fori_loop(0, CPS, body, 0)
    plsc.subcore_barrier()
    pltpu.sync_copy(acc.at[pl.ds(r0, RPT)], out_hbm.at[c, pl.ds(r0, RPT)])


@functools.cache
def _deg_call():
    return pl.kernel(
        _deg_body,
        out_type=jax.ShapeDtypeStruct((2, N_PAD, DEG_W), jnp.float32),
        scratch_types=[
            pltpu.VMEM((CPS, CHUNK), jnp.int32),
            pltpu.VMEM((CHUNK, DEG_W), jnp.float32),
            pltpu.VMEM_SHARED((N_PAD, DEG_W), jnp.float32),
        ],
        mesh=plsc.VectorSubcoreMesh(core_axis_name="c", subcore_axis_name="s"),
    )


def _agg_body(g_hbm, src_hbm, dst_hbm, zeros_hbm, out_hbm,
              src_v, dst_v, rows_a, rows_b, sem_ga, sem_gb, sem_sa, sem_sb,
              acc):
    c = lax.axis_index("c")
    s = lax.axis_index("s")
    wid = c * 16 + s
    r0 = s * RPT
    pltpu.sync_copy(zeros_hbm.at[pl.ds(r0, RPT)], acc.at[pl.ds(r0, RPT)])
    plsc.subcore_barrier()

    npairs = CPS // 2

    for st in range(N_STAGES):
        pltpu.sync_copy(src_hbm.at[wid, st], src_v)
        pltpu.sync_copy(dst_hbm.at[wid, st], dst_v)
        pltpu.async_copy(g_hbm.at[src_v.at[0]], rows_a, sem_ga)
        pltpu.async_copy(g_hbm.at[src_v.at[1]], rows_b, sem_gb)

        def body(gidx, carry):
            ja = 2 * gidx
            jb = ja + 1
            pltpu.make_async_copy(g_hbm.at[src_v.at[ja]], rows_a, sem_ga).wait()
            pltpu.sync_copy(rows_a, acc.at[dst_v.at[ja]], add=True)

            @pl.when(gidx + 1 < npairs)
            def _start_a():
                pltpu.async_copy(g_hbm.at[src_v.at[ja + 2]], rows_a, sem_ga)

            pltpu.make_async_copy(g_hbm.at[src_v.at[jb]], rows_b, sem_gb).wait()
            pltpu.sync_copy(rows_b, acc.at[dst_v.at[jb]], add=True)

            @pl.when(gidx + 1 < npairs)
            def _start_b():
                pltpu.async_copy(g_hbm.at[src_v.at[jb + 2]], rows_b, sem_gb)

            return carry

        lax.fori_loop(0, npairs, body, 0)
    plsc.subcore_barrier()
    pltpu.sync_copy(acc.at[pl.ds(r0, RPT)], out_hbm.at[c, pl.ds(r0, RPT)])


@functools.cache
def _agg_call():
    return pl.kernel(
        _agg_body,
        out_type=jax.ShapeDtypeStruct((2, N_PAD, D), jnp.float32),
        scratch_types=[
            pltpu.VMEM((CPS, CHUNK), jnp.int32),
            pltpu.VMEM((CPS, CHUNK), jnp.int32),
            pltpu.VMEM((CHUNK, D), jnp.float32),
            pltpu.VMEM((CHUNK, D), jnp.float32),
            pltpu.SemaphoreType.DMA,
            pltpu.SemaphoreType.DMA,
            pltpu.SemaphoreType.DMA,
            pltpu.SemaphoreType.DMA,
            pltpu.VMEM_SHARED((N_PAD, D), jnp.float32),
        ],
        mesh=plsc.VectorSubcoreMesh(core_axis_name="c", subcore_axis_name="s"),
    )


def _mm1_body(x_ref, w_ref, o_ref):
    o_ref[...] = jnp.dot(
        x_ref[...], w_ref[...], preferred_element_type=jnp.float32)


_mm1_call = pl.pallas_call(
    _mm1_body,
    grid=(GRID,),
    in_specs=[
        pl.BlockSpec((BM, D), lambda i: (i, 0)),
        pl.BlockSpec((D, D), lambda i: (0, 0)),
    ],
    out_specs=pl.BlockSpec((BM, D), lambda i: (i, 0)),
    out_shape=jax.ShapeDtypeStruct((N_PAD, D), jnp.float32),
)


def _scale1_body(deg_ref, h1_ref, dinv_ref, g1_ref):
    dinv = lax.rsqrt(deg_ref[0] + deg_ref[1] + 1.0)
    dinv_ref[...] = dinv
    g1_ref[...] = dinv * h1_ref[...]


_scale1_call = pl.pallas_call(
    _scale1_body,
    grid=(GRID,),
    in_specs=[
        pl.BlockSpec((2, BM, DEG_W), lambda i: (0, i, 0)),
        pl.BlockSpec((BM, D), lambda i: (i, 0)),
    ],
    out_specs=[
        pl.BlockSpec((BM, D), lambda i: (i, 0)),
        pl.BlockSpec((BM, D), lambda i: (i, 0)),
    ],
    out_shape=[
        jax.ShapeDtypeStruct((N_PAD, D), jnp.float32),
        jax.ShapeDtypeStruct((N_PAD, D), jnp.float32),
    ],
)


def _mid_body(agg_ref, g1_ref, dinv_ref, b1_ref, w2_ref, o_ref):
    t = (dinv_ref[...] * (agg_ref[0] + agg_ref[1] + g1_ref[...])
         + b1_ref[...])
    t = jnp.maximum(t, 0.0)
    o_ref[...] = dinv_ref[...] * jnp.dot(
        t, w2_ref[...], preferred_element_type=jnp.float32)


_mid_call = pl.pallas_call(
    _mid_body,
    grid=(GRID,),
    in_specs=[
        pl.BlockSpec((2, BM, D), lambda i: (0, i, 0)),
        pl.BlockSpec((BM, D), lambda i: (i, 0)),
        pl.BlockSpec((BM, D), lambda i: (i, 0)),
        pl.BlockSpec((1, D), lambda i: (0, 0)),
        pl.BlockSpec((D, D), lambda i: (0, 0)),
    ],
    out_specs=pl.BlockSpec((BM, D), lambda i: (i, 0)),
    out_shape=jax.ShapeDtypeStruct((N_PAD, D), jnp.float32),
)

BMF = 1000


def _fin_body(agg_ref, g2_ref, dinv_ref, b2_ref, o_ref):
    o_ref[...] = (dinv_ref[...] * (agg_ref[0] + agg_ref[1] + g2_ref[...])
                  + b2_ref[...])


_fin_call = pl.pallas_call(
    _fin_body,
    grid=(N // BMF,),
    in_specs=[
        pl.BlockSpec((2, BMF, D), lambda i: (0, i, 0)),
        pl.BlockSpec((BMF, D), lambda i: (i, 0)),
        pl.BlockSpec((BMF, D), lambda i: (i, 0)),
        pl.BlockSpec((1, D), lambda i: (0, 0)),
    ],
    out_specs=pl.BlockSpec((BMF, D), lambda i: (i, 0)),
    out_shape=jax.ShapeDtypeStruct((N, D), jnp.float32),
)


def kernel(x, edge_index, W1, b1, W2, b2):
    ei = edge_index.astype(jnp.int32)
    pad_e = E_PAD - E
    it = jnp.arange(pad_e, dtype=jnp.int32)
    src_p = jnp.concatenate(
        [ei[0], it % N]
    ).reshape(N_TILES, N_STAGES, CPS, CHUNK)
    dst_p = jnp.concatenate(
        [ei[1], DUMMY + it % (N_PAD - N)]
    ).reshape(N_TILES, N_STAGES, CPS, CHUNK)
    x_p = jnp.pad(x, ((0, N_PAD - N), (0, 0)))
    zeros_d = jnp.zeros((N_PAD, D), jnp.float32)

    deg = _deg_call()(dst_p)
    h1 = _mm1_call(x_p, W1)
    dinv, g1 = _scale1_call(deg, h1)
    agg1 = _agg_call()(g1, src_p, dst_p, zeros_d)
    g2 = _mid_call(agg1, g1, dinv, b1.reshape(1, D), W2)
    agg2 = _agg_call()(g2, src_p, dst_p, zeros_d)
    return _fin_call(agg2, g2, dinv, b2.reshape(1, D))


# speedup vs baseline: 1.3022x; 1.0211x over previous
import functools

import numpy as np

import jax
import jax.numpy as jnp
from jax import lax
from jax.experimental import pallas as pl
from jax.experimental.pallas import tpu as pltpu
from jax.experimental.pallas import tpu_sc as plsc

N = 10000
E = 320000
D = 128
N_PAD = 10240
N_TILES = 32
CHUNK = 128
CPT = 80
E_PAD = N_TILES * CPT * CHUNK
N_STAGES = 2
CPS = CPT // N_STAGES
DUMMY = N
RPT = N_PAD // 16
DEG_W = 128
BM = 512
GRID = N_PAD // BM


def _deg_body(dst_hbm, out_hbm, dst_v, ones_v, acc):
    c = lax.axis_index("c")
    s = lax.axis_index("s")
    wid = c * 16 + s
    r0 = s * RPT

    def fill0(i, carry):
        ones_v[i] = jnp.zeros((DEG_W,), jnp.float32)
        return carry

    lax.fori_loop(0, CHUNK, fill0, 0)
    for t in range(RPT // CHUNK):
        pltpu.sync_copy(ones_v, acc.at[pl.ds(r0 + t * CHUNK, CHUNK)])

    def fill1(i, carry):
        ones_v[i] = jnp.full((DEG_W,), 1.0, jnp.float32)
        return carry

    lax.fori_loop(0, CHUNK, fill1, 0)
    plsc.subcore_barrier()

    for st in range(N_STAGES):
        pltpu.sync_copy(dst_hbm.at[wid, st], dst_v)

        def body(j, carry):
            pltpu.sync_copy(ones_v, acc.at[dst_v.at[j]], add=True)
            return carry

        lax.fori_loop(0, CPS, body, 0)
    plsc.subcore_barrier()
    pltpu.sync_copy(acc.at[pl.ds(r0, RPT)], out_hbm.at[c, pl.ds(r0, RPT)])


@functools.cache
def _deg_call():
    return pl.kernel(
        _deg_body,
        out_type=jax.ShapeDtypeStruct((2, N_PAD, DEG_W), jnp.float32),
        scratch_types=[
            pltpu.VMEM((CPS, CHUNK), jnp.int32),
            pltpu.VMEM((CHUNK, DEG_W), jnp.float32),
            pltpu.VMEM_SHARED((N_PAD, DEG_W), jnp.float32),
        ],
        mesh=plsc.VectorSubcoreMesh(core_axis_name="c", subcore_axis_name="s"),
    )


def _agg_body(g_hbm, src_hbm, dst_hbm, out_hbm,
              src_v, dst_v, rows_a, rows_b, sem_ga, sem_gb, acc):
    c = lax.axis_index("c")
    s = lax.axis_index("s")
    wid = c * 16 + s
    r0 = s * RPT

    def fill0(i, carry):
        rows_a[i] = jnp.zeros((D,), jnp.float32)
        return carry

    lax.fori_loop(0, CHUNK, fill0, 0)
    for t in range(RPT // CHUNK):
        pltpu.sync_copy(rows_a, acc.at[pl.ds(r0 + t * CHUNK, CHUNK)])
    plsc.subcore_barrier()

    npairs = CPS // 2

    for st in range(N_STAGES):
        pltpu.sync_copy(src_hbm.at[wid, st], src_v)
        pltpu.sync_copy(dst_hbm.at[wid, st], dst_v)
        pltpu.async_copy(g_hbm.at[src_v.at[0]], rows_a, sem_ga)
        pltpu.async_copy(g_hbm.at[src_v.at[1]], rows_b, sem_gb)

        def body(gidx, carry):
            ja = 2 * gidx
            jb = ja + 1
            pltpu.make_async_copy(g_hbm.at[src_v.at[ja]], rows_a, sem_ga).wait()
            pltpu.sync_copy(rows_a, acc.at[dst_v.at[ja]], add=True)

            @pl.when(gidx + 1 < npairs)
            def _start_a():
                pltpu.async_copy(g_hbm.at[src_v.at[ja + 2]], rows_a, sem_ga)

            pltpu.make_async_copy(g_hbm.at[src_v.at[jb]], rows_b, sem_gb).wait()
            pltpu.sync_copy(rows_b, acc.at[dst_v.at[jb]], add=True)

            @pl.when(gidx + 1 < npairs)
            def _start_b():
                pltpu.async_copy(g_hbm.at[src_v.at[jb + 2]], rows_b, sem_gb)

            return carry

        lax.fori_loop(0, npairs, body, 0)
    plsc.subcore_barrier()
    pltpu.sync_copy(acc.at[pl.ds(r0, RPT)], out_hbm.at[c, pl.ds(r0, RPT)])


@functools.cache
def _agg_call():
    return pl.kernel(
        _agg_body,
        out_type=jax.ShapeDtypeStruct((2, N_PAD, D), jnp.float32),
        scratch_types=[
            pltpu.VMEM((CPS, CHUNK), jnp.int32),
            pltpu.VMEM((CPS, CHUNK), jnp.int32),
            pltpu.VMEM((CHUNK, D), jnp.float32),
            pltpu.VMEM((CHUNK, D), jnp.float32),
            pltpu.SemaphoreType.DMA,
            pltpu.SemaphoreType.DMA,
            pltpu.VMEM_SHARED((N_PAD, D), jnp.float32),
        ],
        mesh=plsc.VectorSubcoreMesh(core_axis_name="c", subcore_axis_name="s"),
    )


def _mm1_body(x_ref, w_ref, o_ref):
    o_ref[...] = jnp.dot(
        x_ref[...], w_ref[...], preferred_element_type=jnp.float32)


_mm1_call = pl.pallas_call(
    _mm1_body,
    grid=(GRID,),
    in_specs=[
        pl.BlockSpec((BM, D), lambda i: (i, 0)),
        pl.BlockSpec((D, D), lambda i: (0, 0)),
    ],
    out_specs=pl.BlockSpec((BM, D), lambda i: (i, 0)),
    out_shape=jax.ShapeDtypeStruct((N_PAD, D), jnp.float32),
)


def _scale1_body(deg_ref, h1_ref, dinv_ref, g1_ref):
    dinv = lax.rsqrt(deg_ref[0] + deg_ref[1] + 1.0)
    dinv_ref[...] = dinv
    g1_ref[...] = dinv * h1_ref[...]


_scale1_call = pl.pallas_call(
    _scale1_body,
    grid=(GRID,),
    in_specs=[
        pl.BlockSpec((2, BM, DEG_W), lambda i: (0, i, 0)),
        pl.BlockSpec((BM, D), lambda i: (i, 0)),
    ],
    out_specs=[
        pl.BlockSpec((BM, D), lambda i: (i, 0)),
        pl.BlockSpec((BM, D), lambda i: (i, 0)),
    ],
    out_shape=[
        jax.ShapeDtypeStruct((N_PAD, D), jnp.float32),
        jax.ShapeDtypeStruct((N_PAD, D), jnp.float32),
    ],
)


def _mid_body(agg_ref, g1_ref, dinv_ref, b1_ref, w2_ref, o_ref):
    t = (dinv_ref[...] * (agg_ref[0] + agg_ref[1] + g1_ref[...])
         + b1_ref[...])
    t = jnp.maximum(t, 0.0)
    o_ref[...] = dinv_ref[...] * jnp.dot(
        t, w2_ref[...], preferred_element_type=jnp.float32)


_mid_call = pl.pallas_call(
    _mid_body,
    grid=(GRID,),
    in_specs=[
        pl.BlockSpec((2, BM, D), lambda i: (0, i, 0)),
        pl.BlockSpec((BM, D), lambda i: (i, 0)),
        pl.BlockSpec((BM, D), lambda i: (i, 0)),
        pl.BlockSpec((1, D), lambda i: (0, 0)),
        pl.BlockSpec((D, D), lambda i: (0, 0)),
    ],
    out_specs=pl.BlockSpec((BM, D), lambda i: (i, 0)),
    out_shape=jax.ShapeDtypeStruct((N_PAD, D), jnp.float32),
)

BMF = 1000


def _fin_body(agg_ref, g2_ref, dinv_ref, b2_ref, o_ref):
    o_ref[...] = (dinv_ref[...] * (agg_ref[0] + agg_ref[1] + g2_ref[...])
                  + b2_ref[...])


_fin_call = pl.pallas_call(
    _fin_body,
    grid=(N // BMF,),
    in_specs=[
        pl.BlockSpec((2, BMF, D), lambda i: (0, i, 0)),
        pl.BlockSpec((BMF, D), lambda i: (i, 0)),
        pl.BlockSpec((BMF, D), lambda i: (i, 0)),
        pl.BlockSpec((1, D), lambda i: (0, 0)),
    ],
    out_specs=pl.BlockSpec((BMF, D), lambda i: (i, 0)),
    out_shape=jax.ShapeDtypeStruct((N, D), jnp.float32),
)


def kernel(x, edge_index, W1, b1, W2, b2):
    ei = edge_index.astype(jnp.int32)
    pad_e = E_PAD - E
    it = np.arange(pad_e, dtype=np.int32)
    src_p = jnp.concatenate(
        [ei[0], jnp.asarray(it % N)]
    ).reshape(N_TILES, N_STAGES, CPS, CHUNK)
    dst_p = jnp.concatenate(
        [ei[1], jnp.asarray(DUMMY + it % (N_PAD - N))]
    ).reshape(N_TILES, N_STAGES, CPS, CHUNK)
    x_p = jnp.pad(x, ((0, N_PAD - N), (0, 0)))

    deg = _deg_call()(dst_p)
    h1 = _mm1_call(x_p, W1)
    dinv, g1 = _scale1_call(deg, h1)
    agg1 = _agg_call()(g1, src_p, dst_p)
    g2 = _mid_call(agg1, g1, dinv, b1.reshape(1, D), W2)
    agg2 = _agg_call()(g2, src_p, dst_p)
    return _fin_call(agg2, g2, dinv, b2.reshape(1, D))


# speedup vs baseline: 1.3445x; 1.0325x over previous
import functools

import numpy as np

import jax
import jax.numpy as jnp
from jax import lax
from jax.experimental import pallas as pl
from jax.experimental.pallas import tpu as pltpu
from jax.experimental.pallas import tpu_sc as plsc

N = 10000
E = 320000
D = 128
N_PAD = 10240
N_TILES = 32
CHUNK = 128
CPT = 80
E_PAD = N_TILES * CPT * CHUNK
N_STAGES = 2
CPS = CPT // N_STAGES
DUMMY = N
RPT = N_PAD // 16
DEG_W = 128
BM = 1024
GRID = N_PAD // BM


def _deg_body(dst_hbm, out_hbm, dst_v, ones_v, sem_s, acc):
    c = lax.axis_index("c")
    s = lax.axis_index("s")
    wid = c * 16 + s
    r0 = s * RPT

    def fill0(i, carry):
        ones_v[i] = jnp.zeros((DEG_W,), jnp.float32)
        return carry

    lax.fori_loop(0, CHUNK, fill0, 0)
    for t in range(RPT // CHUNK):
        pltpu.sync_copy(ones_v, acc.at[pl.ds(r0 + t * CHUNK, CHUNK)])

    def fill1(i, carry):
        ones_v[i] = jnp.full((DEG_W,), 1.0, jnp.float32)
        return carry

    lax.fori_loop(0, CHUNK, fill1, 0)
    plsc.subcore_barrier()

    for st in range(N_STAGES):
        pltpu.sync_copy(dst_hbm.at[wid, st], dst_v)

        def body(j, carry):
            pltpu.async_copy(ones_v, acc.at[dst_v.at[j]], sem_s, add=True)
            return carry

        lax.fori_loop(0, CPS, body, 0)

        def drain(j, carry):
            pltpu.make_async_copy(ones_v, acc.at[dst_v.at[0]], sem_s).wait()
            return carry

        lax.fori_loop(0, CPS, drain, 0)
    plsc.subcore_barrier()
    pltpu.sync_copy(acc.at[pl.ds(r0, RPT)], out_hbm.at[c, pl.ds(r0, RPT)])


@functools.cache
def _deg_call():
    return pl.kernel(
        _deg_body,
        out_type=jax.ShapeDtypeStruct((2, N_PAD, DEG_W), jnp.float32),
        scratch_types=[
            pltpu.VMEM((CPS, CHUNK), jnp.int32),
            pltpu.VMEM((CHUNK, DEG_W), jnp.float32),
            pltpu.SemaphoreType.DMA,
            pltpu.VMEM_SHARED((N_PAD, DEG_W), jnp.float32),
        ],
        mesh=plsc.VectorSubcoreMesh(core_axis_name="c", subcore_axis_name="s"),
    )


def _agg_body(g_hbm, src_hbm, dst_hbm, out_hbm,
              src_v, dst_v, rows_a, rows_b, sem_ga, sem_gb, acc):
    c = lax.axis_index("c")
    s = lax.axis_index("s")
    wid = c * 16 + s
    r0 = s * RPT

    def fill0(i, carry):
        rows_a[i] = jnp.zeros((D,), jnp.float32)
        return carry

    lax.fori_loop(0, CHUNK, fill0, 0)
    for t in range(RPT // CHUNK):
        pltpu.sync_copy(rows_a, acc.at[pl.ds(r0 + t * CHUNK, CHUNK)])
    plsc.subcore_barrier()

    npairs = CPS // 2

    for st in range(N_STAGES):
        pltpu.sync_copy(src_hbm.at[wid, st], src_v)
        pltpu.sync_copy(dst_hbm.at[wid, st], dst_v)
        pltpu.async_copy(g_hbm.at[src_v.at[0]], rows_a, sem_ga)
        pltpu.async_copy(g_hbm.at[src_v.at[1]], rows_b, sem_gb)

        def body(gidx, carry):
            ja = 2 * gidx
            jb = ja + 1
            pltpu.make_async_copy(g_hbm.at[src_v.at[ja]], rows_a, sem_ga).wait()
            pltpu.sync_copy(rows_a, acc.at[dst_v.at[ja]], add=True)

            @pl.when(gidx + 1 < npairs)
            def _start_a():
                pltpu.async_copy(g_hbm.at[src_v.at[ja + 2]], rows_a, sem_ga)

            pltpu.make_async_copy(g_hbm.at[src_v.at[jb]], rows_b, sem_gb).wait()
            pltpu.sync_copy(rows_b, acc.at[dst_v.at[jb]], add=True)

            @pl.when(gidx + 1 < npairs)
            def _start_b():
                pltpu.async_copy(g_hbm.at[src_v.at[jb + 2]], rows_b, sem_gb)

            return carry

        lax.fori_loop(0, npairs, body, 0)
    plsc.subcore_barrier()
    pltpu.sync_copy(acc.at[pl.ds(r0, RPT)], out_hbm.at[c, pl.ds(r0, RPT)])


@functools.cache
def _agg_call():
    return pl.kernel(
        _agg_body,
        out_type=jax.ShapeDtypeStruct((2, N_PAD, D), jnp.float32),
        scratch_types=[
            pltpu.VMEM((CPS, CHUNK), jnp.int32),
            pltpu.VMEM((CPS, CHUNK), jnp.int32),
            pltpu.VMEM((CHUNK, D), jnp.float32),
            pltpu.VMEM((CHUNK, D), jnp.float32),
            pltpu.SemaphoreType.DMA,
            pltpu.SemaphoreType.DMA,
            pltpu.VMEM_SHARED((N_PAD, D), jnp.float32),
        ],
        mesh=plsc.VectorSubcoreMesh(core_axis_name="c", subcore_axis_name="s"),
    )


def _mm1_body(x_ref, w_ref, o_ref):
    o_ref[...] = jnp.dot(
        x_ref[...], w_ref[...], preferred_element_type=jnp.float32)


_mm1_call = pl.pallas_call(
    _mm1_body,
    grid=(GRID,),
    in_specs=[
        pl.BlockSpec((BM, D), lambda i: (i, 0)),
        pl.BlockSpec((D, D), lambda i: (0, 0)),
    ],
    out_specs=pl.BlockSpec((BM, D), lambda i: (i, 0)),
    out_shape=jax.ShapeDtypeStruct((N_PAD, D), jnp.float32),
)


def _scale1_body(deg_ref, h1_ref, dinv_ref, g1_ref):
    dinv = lax.rsqrt(deg_ref[0] + deg_ref[1] + 1.0)
    dinv_ref[...] = dinv
    g1_ref[...] = dinv * h1_ref[...]


_scale1_call = pl.pallas_call(
    _scale1_body,
    grid=(GRID,),
    in_specs=[
        pl.BlockSpec((2, BM, DEG_W), lambda i: (0, i, 0)),
        pl.BlockSpec((BM, D), lambda i: (i, 0)),
    ],
    out_specs=[
        pl.BlockSpec((BM, D), lambda i: (i, 0)),
        pl.BlockSpec((BM, D), lambda i: (i, 0)),
    ],
    out_shape=[
        jax.ShapeDtypeStruct((N_PAD, D), jnp.float32),
        jax.ShapeDtypeStruct((N_PAD, D), jnp.float32),
    ],
)


def _mid_body(agg_ref, g1_ref, dinv_ref, b1_ref, w2_ref, o_ref):
    t = (dinv_ref[...] * (agg_ref[0] + agg_ref[1] + g1_ref[...])
         + b1_ref[...])
    t = jnp.maximum(t, 0.0)
    o_ref[...] = dinv_ref[...] * jnp.dot(
        t, w2_ref[...], preferred_element_type=jnp.float32)


_mid_call = pl.pallas_call(
    _mid_body,
    grid=(GRID,),
    in_specs=[
        pl.BlockSpec((2, BM, D), lambda i: (0, i, 0)),
        pl.BlockSpec((BM, D), lambda i: (i, 0)),
        pl.BlockSpec((BM, D), lambda i: (i, 0)),
        pl.BlockSpec((1, D), lambda i: (0, 0)),
        pl.BlockSpec((D, D), lambda i: (0, 0)),
    ],
    out_specs=pl.BlockSpec((BM, D), lambda i: (i, 0)),
    out_shape=jax.ShapeDtypeStruct((N_PAD, D), jnp.float32),
)

BMF = 1000


def _fin_body(agg_ref, g2_ref, dinv_ref, b2_ref, o_ref):
    o_ref[...] = (dinv_ref[...] * (agg_ref[0] + agg_ref[1] + g2_ref[...])
                  + b2_ref[...])


_fin_call = pl.pallas_call(
    _fin_body,
    grid=(N // BMF,),
    in_specs=[
        pl.BlockSpec((2, BMF, D), lambda i: (0, i, 0)),
        pl.BlockSpec((BMF, D), lambda i: (i, 0)),
        pl.BlockSpec((BMF, D), lambda i: (i, 0)),
        pl.BlockSpec((1, D), lambda i: (0, 0)),
    ],
    out_specs=pl.BlockSpec((BMF, D), lambda i: (i, 0)),
    out_shape=jax.ShapeDtypeStruct((N, D), jnp.float32),
)


def kernel(x, edge_index, W1, b1, W2, b2):
    ei = edge_index.astype(jnp.int32)
    pad_e = E_PAD - E
    it = np.arange(pad_e, dtype=np.int32)
    src_p = jnp.concatenate(
        [ei[0], jnp.asarray(it % N)]
    ).reshape(N_TILES, N_STAGES, CPS, CHUNK)
    dst_p = jnp.concatenate(
        [ei[1], jnp.asarray(DUMMY + it % (N_PAD - N))]
    ).reshape(N_TILES, N_STAGES, CPS, CHUNK)
    x_p = jnp.pad(x, ((0, N_PAD - N), (0, 0)))

    deg = _deg_call()(dst_p)
    h1 = _mm1_call(x_p, W1)
    dinv, g1 = _scale1_call(deg, h1)
    agg1 = _agg_call()(g1, src_p, dst_p)
    g2 = _mid_call(agg1, g1, dinv, b1.reshape(1, D), W2)
    agg2 = _agg_call()(g2, src_p, dst_p)
    return _fin_call(agg2, g2, dinv, b2.reshape(1, D))
